# baseline (speedup 1.0000x reference)
import functools

import numpy as np

import jax
import jax.numpy as jnp
from jax.experimental import pallas as pl
from jax.experimental.pallas import tpu as pltpu

B = 16384
EMB = 10
NC = 16
SP = 20
SPPAD = 24
BM = 1024

_j = np.arange(200)
_e_of_j = _j // SP
_s_of_j = _j % SP
_c = np.arange(SPPAD * NC)
_k_of_c = _c % NC
_s_of_c = (_c // 128) * 8 + (_c % 128) // NC
_E = np.zeros((200, EMB), np.float32)
_E[_j, _e_of_j] = 1.0
_K = np.zeros((NC, SPPAD * NC), np.float32)
_K[_k_of_c, _c] = 1.0
_S = (_s_of_j[:, None] == _s_of_c[None, :]).astype(np.float32)
_ST = np.ascontiguousarray(_S.T)


def _vq_body(x_ref, w1_ref, b1_ref, w2_ref, b2_ref, m_ref, c2_ref, p_ref,
             w3_ref, b3_ref, w4_ref, b4_ref, recon_ref, ze_ref, zq_ref):
    h1 = jnp.maximum(
        jnp.dot(x_ref[...], w1_ref[...], preferred_element_type=jnp.float32)
        + b1_ref[...], 0.0)
    h2 = jnp.dot(h1, w2_ref[...], preferred_element_type=jnp.float32) + b2_ref[...]
    ze_ref[...] = h2.reshape(h2.shape[0], EMB, SP)

    d = jnp.dot(h2, m_ref[...], preferred_element_type=jnp.float32) + c2_ref[...]
    bm = d.shape[0]
    off = jnp.bitwise_and(
        jax.lax.broadcasted_iota(jnp.int32, (bm, 128), 1), NC - 1)

    parts = []
    for t in range(3):
        dt = jax.lax.slice(d, (0, t * 128), (bm, (t + 1) * 128))
        bits = jax.lax.bitcast_convert_type(dt, jnp.int32)
        key = jax.lax.bitwise_xor(
            bits, jax.lax.bitwise_and(
                jax.lax.shift_right_arithmetic(bits, 31),
                jnp.int32(0x7FFFFFFF)))
        keyk = jax.lax.bitwise_or(
            jax.lax.bitwise_and(key, jnp.int32(~(NC - 1))), off)
        for sh in (8, 4, 2, 1):
            fwd = jnp.roll(keyk, -sh, axis=1)
            bwd = jnp.roll(keyk, sh, axis=1)
            keyk = jnp.minimum(
                keyk, jnp.where(jnp.bitwise_and(off, sh) == 0, fwd, bwd))
        win = jnp.bitwise_and(keyk, NC - 1)
        parts.append((off == win).astype(jnp.float32))
    onehot = jnp.concatenate(parts, axis=1)
    quant = jnp.dot(onehot, p_ref[...], preferred_element_type=jnp.float32)
    zq_ref[...] = quant

    h3 = jnp.maximum(
        jnp.dot(quant, w3_ref[...], preferred_element_type=jnp.float32)
        + b3_ref[...], 0.0)
    recon_ref[...] = jax.nn.sigmoid(
        jnp.dot(h3, w4_ref[...], preferred_element_type=jnp.float32) + b4_ref[...])


@functools.partial(jax.jit, static_argnames=())
def _run(x, W1T, b1, W2T, b2, M, c2, P, W3T, b3, W4T, b4):
    grid = (B // BM,)
    row_blk = lambda shp: pl.BlockSpec(shp, lambda i: (i, 0))
    full_blk = lambda shp: pl.BlockSpec(shp, lambda i: (0, 0))
    out_shapes = (
        jax.ShapeDtypeStruct((B, 784), jnp.float32),
        jax.ShapeDtypeStruct((B, EMB, SP), jnp.float32),
        jax.ShapeDtypeStruct((B, 200), jnp.float32),
    )
    return pl.pallas_call(
        _vq_body,
        grid=grid,
        in_specs=[
            row_blk((BM, 784)),
            full_blk((784, 400)), full_blk((1, 400)),
            full_blk((400, 200)), full_blk((1, 200)),
            full_blk((200, SPPAD * NC)), full_blk((1, SPPAD * NC)),
            full_blk((SPPAD * NC, 200)),
            full_blk((200, 400)), full_blk((1, 400)),
            full_blk((400, 784)), full_blk((1, 784)),
        ],
        out_specs=(
            row_blk((BM, 784)),
            pl.BlockSpec((BM, EMB, SP), lambda i: (i, 0, 0)),
            row_blk((BM, 200)),
        ),
        out_shape=out_shapes,
        compiler_params=pltpu.CompilerParams(
            dimension_semantics=("arbitrary",)),
    )(x, W1T, b1, W2T, b2, M, c2, P, W3T, b3, W4T, b4)


def kernel(x, W1, b1, W2, b2, W3, b3, W4, b4, codebook):
    hi = jax.lax.Precision.HIGHEST
    dot = lambda a, b: jax.lax.dot(a, b, precision=hi)
    M = _S * dot(dot(_E, -2.0 * codebook.T), _K)
    c2 = dot(jnp.sum(codebook * codebook, axis=1)[None, :], _K)
    P = _ST * dot(dot(_K.T, codebook), _E.T)

    recon, ze, zq_flat = _run(
        x, W1.T, b1[None, :], W2.T, b2[None, :], M, c2, P,
        W3.T, b3[None, :], W4.T, b4[None, :])
    return (recon, ze, zq_flat)


# speedup vs baseline: 1.2834x; 1.2834x over previous
import functools

import numpy as np

import jax
import jax.numpy as jnp
from jax.experimental import pallas as pl
from jax.experimental.pallas import tpu as pltpu

B = 16384
EMB = 10
NC = 16
SP = 20
SPPAD = 24
BM = 1024

_j = np.arange(200)
_e_of_j = _j // SP
_s_of_j = _j % SP
_c = np.arange(SPPAD * NC)
_k_of_c = _c % NC
_s_of_c = (_c // 128) * 8 + (_c % 128) // NC
_E = np.zeros((200, EMB), np.float32)
_E[_j, _e_of_j] = 1.0
_K = np.zeros((NC, SPPAD * NC), np.float32)
_K[_k_of_c, _c] = 1.0
_S = (_s_of_j[:, None] == _s_of_c[None, :]).astype(np.float32)
_ST = np.ascontiguousarray(_S.T)


def _dott(a, w):
    return jax.lax.dot_general(
        a, w, (((1,), (1,)), ((), ())), preferred_element_type=jnp.float32)


def _vq_body(x_ref, w1_ref, b1_ref, w2_ref, b2_ref, m_ref, c2_ref, p_ref,
             w3_ref, b3_ref, w4_ref, b4_ref, recon_ref, ze_ref, zq_ref):
    h1 = jnp.maximum(_dott(x_ref[...], w1_ref[...]) + b1_ref[...], 0.0)
    h2 = _dott(h1, w2_ref[...]) + b2_ref[...]
    ze_ref[...] = h2

    d = jnp.dot(h2, m_ref[...], preferred_element_type=jnp.float32) + c2_ref[...]
    bm = d.shape[0]
    off = jnp.bitwise_and(
        jax.lax.broadcasted_iota(jnp.int32, (bm, 128), 1), NC - 1)

    parts = []
    for t in range(3):
        dt = jax.lax.slice(d, (0, t * 128), (bm, (t + 1) * 128))
        bits = jax.lax.bitcast_convert_type(dt, jnp.int32)
        key = jax.lax.bitwise_xor(
            bits, jax.lax.bitwise_and(
                jax.lax.shift_right_arithmetic(bits, 31),
                jnp.int32(0x7FFFFFFF)))
        keyk = jax.lax.bitwise_or(
            jax.lax.bitwise_and(key, jnp.int32(~(NC - 1))), off)
        for sh in (8, 4, 2, 1):
            fwd = jnp.roll(keyk, -sh, axis=1)
            bwd = jnp.roll(keyk, sh, axis=1)
            keyk = jnp.minimum(
                keyk, jnp.where(jnp.bitwise_and(off, sh) == 0, fwd, bwd))
        win = jnp.bitwise_and(keyk, NC - 1)
        parts.append((off == win).astype(jnp.float32))
    onehot = jnp.concatenate(parts, axis=1)
    quant = jnp.dot(onehot, p_ref[...], preferred_element_type=jnp.float32)
    zq_ref[...] = quant

    h3 = jnp.maximum(_dott(quant, w3_ref[...]) + b3_ref[...], 0.0)
    recon_ref[...] = jax.nn.sigmoid(_dott(h3, w4_ref[...]) + b4_ref[...])


@functools.partial(jax.jit, static_argnames=())
def _run(x, W1T, b1, W2T, b2, M, c2, P, W3T, b3, W4T, b4):
    grid = (B // BM,)
    row_blk = lambda shp: pl.BlockSpec(shp, lambda i: (i, 0))
    full_blk = lambda shp: pl.BlockSpec(shp, lambda i: (0, 0))
    out_shapes = (
        jax.ShapeDtypeStruct((B, 784), jnp.float32),
        jax.ShapeDtypeStruct((B, 200), jnp.float32),
        jax.ShapeDtypeStruct((B, 200), jnp.float32),
    )
    return pl.pallas_call(
        _vq_body,
        grid=grid,
        in_specs=[
            row_blk((BM, 784)),
            full_blk((400, 784)), full_blk((1, 400)),
            full_blk((200, 400)), full_blk((1, 200)),
            full_blk((200, SPPAD * NC)), full_blk((1, SPPAD * NC)),
            full_blk((SPPAD * NC, 200)),
            full_blk((400, 200)), full_blk((1, 400)),
            full_blk((784, 400)), full_blk((1, 784)),
        ],
        out_specs=(row_blk((BM, 784)), row_blk((BM, 200)), row_blk((BM, 200))),
        out_shape=out_shapes,
        compiler_params=pltpu.CompilerParams(
            dimension_semantics=("arbitrary",)),
    )(x, W1T, b1, W2T, b2, M, c2, P, W3T, b3, W4T, b4)


def kernel(x, W1, b1, W2, b2, W3, b3, W4, b4, codebook):
    hi = jax.lax.Precision.HIGHEST
    dot = lambda a, b: jax.lax.dot(a, b, precision=hi)
    M = _S * dot(dot(_E, -2.0 * codebook.T), _K)
    c2 = dot(jnp.sum(codebook * codebook, axis=1)[None, :], _K)
    P = _ST * dot(dot(_K.T, codebook), _E.T)

    recon, ze_flat, zq_flat = _run(
        x, W1, b1[None, :], W2, b2[None, :], M, c2, P,
        W3, b3[None, :], W4, b4[None, :])
    return (recon, ze_flat.reshape(B, EMB, SP), zq_flat)


# speedup vs baseline: 3.2205x; 2.5094x over previous
import functools

import numpy as np

import jax
import jax.numpy as jnp
from jax.experimental import pallas as pl
from jax.experimental.pallas import tpu as pltpu

B = 16384
EMB = 10
NC = 16
SP = 20
NCOL = SP * NC
BM = 1024

_j = np.arange(200)
_e_of_j = _j // SP
_s_of_j = _j % SP
_c = np.arange(NCOL)
_k_of_c = _c % NC
_s_of_c = _c // NC
_E = np.zeros((200, EMB), np.float32)
_E[_j, _e_of_j] = 1.0
_K = np.zeros((NC, NCOL), np.float32)
_K[_k_of_c, _c] = 1.0
_ST = np.ascontiguousarray(
    (_s_of_c[:, None] == _s_of_j[None, :]).astype(np.float32))


def _body(xt_ref, w1_ref, b1_ref, w2_ref, b2_ref, mt_ref, c2_ref, pt_ref,
          w3_ref, b3_ref, w4_ref, b4_ref, recont_ref, zet_ref, zqt_ref):
    f32 = jnp.float32
    h1t = jnp.maximum(
        jnp.dot(w1_ref[...], xt_ref[...], preferred_element_type=f32)
        + b1_ref[...], 0.0)
    h2t = jnp.dot(w2_ref[...], h1t, preferred_element_type=f32) + b2_ref[...]
    zet_ref[...] = h2t

    dt = jnp.dot(mt_ref[...], h2t, preferred_element_type=f32) + c2_ref[...]
    bm = dt.shape[1]
    off = jnp.bitwise_and(
        jax.lax.broadcasted_iota(jnp.int32, (NCOL, bm), 0), NC - 1)

    bits = jax.lax.bitcast_convert_type(dt, jnp.int32)
    key = jax.lax.bitwise_xor(
        bits, jax.lax.bitwise_and(
            jax.lax.shift_right_arithmetic(bits, 31), jnp.int32(0x7FFFFFFF)))
    keyk = jax.lax.bitwise_or(
        jax.lax.bitwise_and(key, jnp.int32(~(NC - 1))), off)
    for sh in (8, 4, 2, 1):
        fwd = jnp.roll(keyk, -sh, axis=0)
        bwd = jnp.roll(keyk, sh, axis=0)
        keyk = jnp.minimum(
            keyk, jnp.where(jnp.bitwise_and(off, sh) == 0, fwd, bwd))
    win = jnp.bitwise_and(keyk, NC - 1)
    onehot = (off == win).astype(f32)

    quant_t = jnp.dot(pt_ref[...], onehot, preferred_element_type=f32)
    zqt_ref[...] = quant_t

    h3t = jnp.maximum(
        jnp.dot(w3_ref[...], quant_t, preferred_element_type=f32)
        + b3_ref[...], 0.0)
    recont_ref[...] = jax.nn.sigmoid(
        jnp.dot(w4_ref[...], h3t, preferred_element_type=f32) + b4_ref[...])


@functools.partial(jax.jit, static_argnames=())
def _run(xt, W1, b1, W2, b2, MT, c2, PT, W3, b3, W4, b4):
    grid = (B // BM,)
    col_blk = lambda shp: pl.BlockSpec(shp, lambda i: (0, i))
    full_blk = lambda shp: pl.BlockSpec(shp, lambda i: (0, 0))
    out_shapes = (
        jax.ShapeDtypeStruct((784, B), jnp.float32),
        jax.ShapeDtypeStruct((200, B), jnp.float32),
        jax.ShapeDtypeStruct((200, B), jnp.float32),
    )
    return pl.pallas_call(
        _body,
        grid=grid,
        in_specs=[
            col_blk((784, BM)),
            full_blk((400, 784)), full_blk((400, 1)),
            full_blk((200, 400)), full_blk((200, 1)),
            full_blk((NCOL, 200)), full_blk((NCOL, 1)),
            full_blk((200, NCOL)),
            full_blk((400, 200)), full_blk((400, 1)),
            full_blk((784, 400)), full_blk((784, 1)),
        ],
        out_specs=(col_blk((784, BM)), col_blk((200, BM)), col_blk((200, BM))),
        out_shape=out_shapes,
        compiler_params=pltpu.CompilerParams(
            dimension_semantics=("arbitrary",)),
    )(xt, W1, b1, W2, b2, MT, c2, PT, W3, b3, W4, b4)


def kernel(x, W1, b1, W2, b2, W3, b3, W4, b4, codebook):
    hi = jax.lax.Precision.HIGHEST
    dot = lambda a, b: jax.lax.dot(a, b, precision=hi)
    MT = _ST * dot(dot(_E, -2.0 * codebook.T), _K).T
    c2 = dot(_K.T, jnp.sum(codebook * codebook, axis=1)[:, None])
    PT = _ST.T * dot(dot(_E, codebook.T), _K)

    recont, zet, zqt = _run(
        x.T, W1, b1[:, None], W2, b2[:, None], MT, c2, PT,
        W3, b3[:, None], W4, b4[:, None])
    return (recont.T, zet.T.reshape(B, EMB, SP), zqt.T)


# speedup vs baseline: 3.5865x; 1.1136x over previous
import functools

import numpy as np

import jax
import jax.numpy as jnp
from jax.experimental import pallas as pl
from jax.experimental.pallas import tpu as pltpu

B = 16384
EMB = 10
NC = 16
SP = 20
NCOL = SP * NC
BM = 1024

_j = np.arange(200)
_e_of_j = _j // SP
_s_of_j = _j % SP
_c = np.arange(NCOL)
_k_of_c = _c % NC
_s_of_c = _c // NC
_E = np.zeros((200, EMB), np.float32)
_E[_j, _e_of_j] = 1.0
_K = np.zeros((NC, NCOL), np.float32)
_K[_k_of_c, _c] = 1.0
_ST = np.ascontiguousarray(
    (_s_of_c[:, None] == _s_of_j[None, :]).astype(np.float32))


def _body(xt_ref, w1_ref, b1_ref, w2_ref, b2_ref, mt_ref, c2_ref, pt_ref,
          w3_ref, b3_ref, w4_ref, b4_ref, recont_ref, zet_ref, zqt_ref):
    f32 = jnp.float32
    h1t = jnp.maximum(
        jnp.dot(w1_ref[...], xt_ref[...], preferred_element_type=f32)
        + b1_ref[...], 0.0)
    h2t = jnp.dot(w2_ref[...], h1t, preferred_element_type=f32) + b2_ref[...]
    for e in range(EMB):
        zet_ref[e, :, :] = jax.lax.slice(
            h2t, (e * SP, 0), ((e + 1) * SP, h2t.shape[1]))

    dt = jnp.dot(mt_ref[...], h2t, preferred_element_type=f32) + c2_ref[...]
    bm = dt.shape[1]
    off = jnp.bitwise_and(
        jax.lax.broadcasted_iota(jnp.int32, (NCOL, bm), 0), NC - 1)

    bits = jax.lax.bitcast_convert_type(dt, jnp.int32)
    key = jax.lax.bitwise_xor(
        bits, jax.lax.bitwise_and(
            jax.lax.shift_right_arithmetic(bits, 31), jnp.int32(0x7FFFFFFF)))
    keyk = jax.lax.bitwise_or(
        jax.lax.bitwise_and(key, jnp.int32(~(NC - 1))), off)
    for sh in (8, 4, 2, 1):
        fwd = jnp.roll(keyk, -sh, axis=0)
        bwd = jnp.roll(keyk, sh, axis=0)
        keyk = jnp.minimum(
            keyk, jnp.where(jnp.bitwise_and(off, sh) == 0, fwd, bwd))
    win = jnp.bitwise_and(keyk, NC - 1)
    onehot = (off == win).astype(f32)

    quant_t = jnp.dot(pt_ref[...], onehot, preferred_element_type=f32)
    zqt_ref[...] = quant_t

    h3t = jnp.maximum(
        jnp.dot(w3_ref[...], quant_t, preferred_element_type=f32)
        + b3_ref[...], 0.0)
    recont_ref[...] = jax.nn.sigmoid(
        jnp.dot(w4_ref[...], h3t, preferred_element_type=f32) + b4_ref[...])


@functools.partial(jax.jit, static_argnames=())
def _run(xt, W1, b1, W2, b2, MT, c2, PT, W3, b3, W4, b4):
    grid = (B // BM,)
    col_blk = lambda shp: pl.BlockSpec(shp, lambda i: (0, i))
    full_blk = lambda shp: pl.BlockSpec(shp, lambda i: (0, 0))
    out_shapes = (
        jax.ShapeDtypeStruct((784, B), jnp.float32),
        jax.ShapeDtypeStruct((EMB, SP, B), jnp.float32),
        jax.ShapeDtypeStruct((200, B), jnp.float32),
    )
    return pl.pallas_call(
        _body,
        grid=grid,
        in_specs=[
            col_blk((784, BM)),
            full_blk((400, 784)), full_blk((400, 1)),
            full_blk((200, 400)), full_blk((200, 1)),
            full_blk((NCOL, 200)), full_blk((NCOL, 1)),
            full_blk((200, NCOL)),
            full_blk((400, 200)), full_blk((400, 1)),
            full_blk((784, 400)), full_blk((784, 1)),
        ],
        out_specs=(
            col_blk((784, BM)),
            pl.BlockSpec((EMB, SP, BM), lambda i: (0, 0, i)),
            col_blk((200, BM)),
        ),
        out_shape=out_shapes,
        compiler_params=pltpu.CompilerParams(
            dimension_semantics=("arbitrary",)),
    )(xt, W1, b1, W2, b2, MT, c2, PT, W3, b3, W4, b4)


def kernel(x, W1, b1, W2, b2, W3, b3, W4, b4, codebook):
    hi = jax.lax.Precision.HIGHEST
    dot = lambda a, b: jax.lax.dot(a, b, precision=hi)
    MT = _ST * dot(dot(_E, -2.0 * codebook.T), _K).T
    c2 = dot(_K.T, jnp.sum(codebook * codebook, axis=1)[:, None])
    PT = _ST.T * dot(dot(_E, codebook.T), _K)

    recont, ze3, zqt = _run(
        x.T, W1, b1[:, None], W2, b2[:, None], MT, c2, PT,
        W3, b3[:, None], W4, b4[:, None])
    return (recont.T, ze3.transpose(2, 0, 1), zqt.T)
